# baseline (speedup 1.0000x reference)
import functools

import jax
import jax.numpy as jnp
from jax import lax
from jax.experimental import pallas as pl
from jax.experimental.pallas import tpu as pltpu
from jax.experimental.pallas import tpu_sc as plsc

N_NODES = 40962
N_OUT = 10242
K = 7
N_ROWS = 1024
NUM_WORKERS = 32
ROWS_PER_W = N_ROWS // NUM_WORKERS
N_OUT_PAD = ((N_OUT + 15) // 16) * 16
GROUPS = N_OUT_PAD // 16
IDX_PAD = N_OUT_PAD
OUT_STRIDE = ((N_OUT + 7) // 8) * 8
ROW_BUF = ((N_NODES + 7) // 8) * 8


@functools.partial(
    pl.kernel,
    mesh=plsc.VectorSubcoreMesh(core_axis_name="c", subcore_axis_name="s"),
    compiler_params=pltpu.CompilerParams(needs_layout_passes=False),
    out_type=jax.ShapeDtypeStruct((N_ROWS * OUT_STRIDE,), jnp.float32),
    scratch_types=[
        pltpu.VMEM((K * IDX_PAD,), jnp.int32),
        pltpu.VMEM((ROW_BUF,), jnp.float32),
        pltpu.VMEM((N_OUT_PAD,), jnp.float32),
    ],
)
def _pool(x_hbm, idx_hbm, out_hbm, idx_v, row_v, out_v):
    wid = lax.axis_index("s") * 2 + lax.axis_index("c")
    pltpu.sync_copy(idx_hbm, idx_v)
    scale = jnp.float32(1.0 / K)

    def row_step(r, carry):
        row = wid * ROWS_PER_W + r
        off = row * N_NODES
        base = (off // 8) * 8
        delta = off - base
        pltpu.sync_copy(x_hbm.at[pl.ds(base, ROW_BUF)], row_v)

        def grp(g, c2):
            col = g * 16
            acc = plsc.load_gather(row_v, [idx_v[pl.ds(col, 16)] + delta])
            for k in range(1, K):
                ivec = idx_v[pl.ds(k * IDX_PAD + col, 16)] + delta
                acc = acc + plsc.load_gather(row_v, [ivec])
            out_v[pl.ds(col, 16)] = acc * scale
            return c2

        lax.fori_loop(0, GROUPS, grp, 0)
        pltpu.sync_copy(
            out_v.at[pl.ds(0, OUT_STRIDE)], out_hbm.at[pl.ds(row * OUT_STRIDE, OUT_STRIDE)]
        )
        return carry

    lax.fori_loop(0, ROWS_PER_W, row_step, 0)


def kernel(x, neigh_orders):
    B, D, N = x.shape
    idx = neigh_orders[: N_OUT * K].astype(jnp.int32).reshape(N_OUT, K).T
    idx = jnp.pad(idx, ((0, 0), (0, IDX_PAD - N_OUT))).reshape(-1)
    out = _pool(x.reshape(-1), idx)
    out = out.reshape(N_ROWS, OUT_STRIDE)[:, :N_OUT]
    return out.reshape(B, D, N_OUT)


# speedup vs baseline: 2.6392x; 2.6392x over previous
import functools

import jax
import jax.numpy as jnp
from jax import lax
from jax.experimental import pallas as pl
from jax.experimental.pallas import tpu as pltpu
from jax.experimental.pallas import tpu_sc as plsc

N_NODES = 40962
N_OUT = 10242
K = 7
N_ROWS = 1024
NUM_WORKERS = 32
ROWS_PER_W = N_ROWS // NUM_WORKERS
N_OUT_PAD = ((N_OUT + 31) // 32) * 32
GROUPS = N_OUT_PAD // 16
IDX_PAD = N_OUT_PAD
OUT_STRIDE = ((N_OUT + 7) // 8) * 8
ROW_PAD = ((N_NODES + 127) // 128) * 128


@functools.partial(
    pl.kernel,
    mesh=plsc.VectorSubcoreMesh(core_axis_name="c", subcore_axis_name="s"),
    compiler_params=pltpu.CompilerParams(needs_layout_passes=False),
    out_type=jax.ShapeDtypeStruct((N_ROWS * OUT_STRIDE,), jnp.float32),
    scratch_types=[
        pltpu.VMEM((K * IDX_PAD,), jnp.int32),
        pltpu.VMEM((ROW_PAD,), jnp.float32),
        pltpu.VMEM((N_OUT_PAD,), jnp.float32),
    ],
)
def _pool(x_hbm, idx_hbm, out_hbm, idx_v, row_v, out_v):
    wid = lax.axis_index("s") * 2 + lax.axis_index("c")
    pltpu.sync_copy(idx_hbm, idx_v)
    scale = jnp.float32(1.0 / K)

    def row_step(r, carry):
        row = wid * ROWS_PER_W + r
        pltpu.sync_copy(x_hbm.at[pl.ds(row * ROW_PAD, ROW_PAD)], row_v)

        def grp(g, c2):
            col = g * 16
            acc = plsc.load_gather(row_v, [idx_v[pl.ds(col, 16)]])
            for k in range(1, K):
                acc = acc + plsc.load_gather(
                    row_v, [idx_v[pl.ds(k * IDX_PAD + col, 16)]]
                )
            out_v[pl.ds(col, 16)] = acc * scale
            return c2

        lax.fori_loop(0, GROUPS, grp, 0)
        pltpu.sync_copy(
            out_v.at[pl.ds(0, OUT_STRIDE)], out_hbm.at[pl.ds(row * OUT_STRIDE, OUT_STRIDE)]
        )
        return carry

    lax.fori_loop(0, ROWS_PER_W, row_step, 0)


def kernel(x, neigh_orders):
    B, D, N = x.shape
    idx = neigh_orders[: N_OUT * K].astype(jnp.int32).reshape(N_OUT, K).T
    idx = jnp.pad(idx, ((0, 0), (0, IDX_PAD - N_OUT))).reshape(-1)
    xp = jnp.pad(x.reshape(B * D, N), ((0, 0), (0, ROW_PAD - N))).reshape(-1)
    out = _pool(xp, idx)
    out = out.reshape(N_ROWS, OUT_STRIDE)[:, :N_OUT]
    return out.reshape(B, D, N_OUT)
